# Initial kernel scaffold; baseline (speedup 1.0000x reference)
import jax
import jax.numpy as jnp
from jax.experimental import pallas as pl


def kernel(x, Wr, br, W1, b1, W2, b2):
    raise NotImplementedError("write your pallas kernel here")


# speedup vs baseline: 1.9985x; 1.9985x over previous
import functools

import jax
import jax.numpy as jnp
from jax.experimental import pallas as pl
from jax.experimental.pallas import tpu as pltpu

N_EMBD = 768
HIDDEN = 3072
N_EXPERTS = 8
TOP_K = 2
TOKENS = 2048

TB = 256


def _router_body(x_ref, wr_ref, br_ref, cw_ref):
    logits = jnp.dot(x_ref[...], wr_ref[...],
                     preferred_element_type=jnp.float32) + br_ref[...]
    col = jax.lax.broadcasted_iota(jnp.int32, logits.shape, 1)
    m1 = jnp.max(logits, axis=1, keepdims=True)
    is1 = logits >= m1
    e0 = jnp.min(jnp.where(is1, col, N_EXPERTS), axis=1, keepdims=True)
    sel0 = col == e0
    masked = jnp.where(sel0, -jnp.inf, logits)
    m2 = jnp.max(masked, axis=1, keepdims=True)
    is2 = masked >= m2
    e1 = jnp.min(jnp.where(is2, col, N_EXPERTS), axis=1, keepdims=True)
    sel1 = col == e1
    p0 = jnp.ones_like(m1)
    p1 = jnp.exp(m2 - m1)
    denom = p0 + p1
    w0 = p0 / denom
    w1 = p1 / denom
    cw_ref[...] = jnp.where(sel0, w0, jnp.where(sel1, w1, 0.0))


def _moe_body(x_ref, w1_ref, b1_ref, w2_ref, b2_ref, cw_ref, out_ref):
    e = pl.program_id(1)
    h = jnp.dot(x_ref[...], w1_ref[0],
                preferred_element_type=jnp.float32) + b1_ref[0]
    h = 0.5 * h * (1.0 + jax.lax.erf(h * 0.7071067811865476))
    o = jnp.dot(h.astype(jnp.bfloat16), w2_ref[0],
                preferred_element_type=jnp.float32) + b2_ref[0]
    onehot = (jax.lax.broadcasted_iota(jnp.int32, (TB, N_EXPERTS), 1) == e)
    w_col = jnp.sum(jnp.where(onehot, cw_ref[...], 0.0), axis=1, keepdims=True)
    contrib = w_col * o

    @pl.when(e == 0)
    def _():
        out_ref[...] = contrib

    @pl.when(e != 0)
    def _():
        out_ref[...] += contrib


def kernel(x, Wr, br, W1, b1, W2, b2):
    cw = pl.pallas_call(
        _router_body,
        out_shape=jax.ShapeDtypeStruct((TOKENS, N_EXPERTS), jnp.float32),
    )(x, Wr, br)

    xb = x.astype(jnp.bfloat16)
    W1b = W1.astype(jnp.bfloat16)
    W2b = W2.astype(jnp.bfloat16)

    grid = (TOKENS // TB, N_EXPERTS)
    out = pl.pallas_call(
        _moe_body,
        grid=grid,
        in_specs=[
            pl.BlockSpec((TB, N_EMBD), lambda t, e: (t, 0)),
            pl.BlockSpec((1, N_EMBD, HIDDEN), lambda t, e: (e, 0, 0)),
            pl.BlockSpec((1, 1, HIDDEN), lambda t, e: (e, 0, 0)),
            pl.BlockSpec((1, HIDDEN, N_EMBD), lambda t, e: (e, 0, 0)),
            pl.BlockSpec((1, 1, N_EMBD), lambda t, e: (e, 0, 0)),
            pl.BlockSpec((TB, N_EXPERTS), lambda t, e: (t, 0)),
        ],
        out_specs=pl.BlockSpec((TB, N_EMBD), lambda t, e: (t, 0)),
        out_shape=jax.ShapeDtypeStruct((TOKENS, N_EMBD), jnp.float32),
    )(xb, W1b, b1.reshape(N_EXPERTS, 1, HIDDEN), W2b,
      b2.reshape(N_EXPERTS, 1, N_EMBD), cw)
    return out


# speedup vs baseline: 2.7197x; 1.3608x over previous
import functools

import jax
import jax.numpy as jnp
from jax.experimental import pallas as pl
from jax.experimental.pallas import tpu as pltpu
from jax.experimental.pallas import tpu_sc as plsc

N_EMBD = 768
HIDDEN = 3072
N_EXPERTS = 8
TOP_K = 2
TOKENS = 2048

RB = 128
NASSIGN = TOP_K * TOKENS
NBLK = NASSIGN // RB + N_EXPERTS
NBLK_PAD = 64
NROWS = NBLK * RB
SC_W = 128
SC_CHUNK = 384


def _router_body(x_ref, wr_ref, br_ref, w01_ref, posk_ref, bexp_ref, nblk_ref):
    logits = jnp.dot(x_ref[...], wr_ref[...],
                     preferred_element_type=jnp.float32) + br_ref[...]
    col = jax.lax.broadcasted_iota(jnp.int32, logits.shape, 1)
    m1 = jnp.max(logits, axis=1, keepdims=True)
    e0 = jnp.min(jnp.where(logits >= m1, col, N_EXPERTS), axis=1, keepdims=True)
    sel0 = col == e0
    masked = jnp.where(sel0, -jnp.inf, logits)
    m2 = jnp.max(masked, axis=1, keepdims=True)
    e1 = jnp.min(jnp.where(masked >= m2, col, N_EXPERTS), axis=1, keepdims=True)
    sel1 = col == e1
    r = jnp.exp(m2 - m1)
    w0 = 1.0 / (1.0 + r)
    w01_ref[...] = jnp.concatenate([w0, 1.0 - w0], axis=1)

    oh0 = sel0.astype(jnp.bfloat16)
    oh1 = sel1.astype(jnp.bfloat16)
    oh = jnp.concatenate([oh0, oh1], axis=1)
    ri = jax.lax.broadcasted_iota(jnp.int32, (TOKENS, TOKENS), 0)
    ci = jax.lax.broadcasted_iota(jnp.int32, (TOKENS, TOKENS), 1)
    tri = (ri > ci).astype(jnp.bfloat16)
    csum = jnp.dot(tri, oh, preferred_element_type=jnp.float32)
    rank0 = jnp.sum(jnp.where(sel0, csum[:, :N_EXPERTS], 0.0), axis=1,
                    keepdims=True)
    rank1 = jnp.sum(jnp.where(sel1, csum[:, N_EXPERTS:], 0.0), axis=1,
                    keepdims=True)

    count0 = jnp.sum(oh0.astype(jnp.float32), axis=0, keepdims=True)
    count1 = jnp.sum(oh1.astype(jnp.float32), axis=0, keepdims=True)
    count = count0 + count1
    rank1 = rank1 + jnp.sum(jnp.where(sel1, count0, 0.0), axis=1,
                            keepdims=True)

    pblk = jnp.floor((count + 127.0) * (1.0 / 128.0))
    ui = jax.lax.broadcasted_iota(jnp.int32, (N_EXPERTS, N_EXPERTS), 0)
    uj = jax.lax.broadcasted_iota(jnp.int32, (N_EXPERTS, N_EXPERTS), 1)
    triu = (ui < uj).astype(jnp.bfloat16)
    pstart_blk = jnp.dot(pblk.astype(jnp.bfloat16), triu,
                         preferred_element_type=jnp.float32)
    pstart_rows = pstart_blk * float(RB)

    pos0 = jnp.sum(jnp.where(sel0, pstart_rows, 0.0), axis=1,
                   keepdims=True) + rank0
    pos1 = jnp.sum(jnp.where(sel1, pstart_rows, 0.0), axis=1,
                   keepdims=True) + rank1
    posk_ref[...] = jnp.concatenate([pos0, pos1], axis=1).astype(jnp.int32)

    bi = jax.lax.broadcasted_iota(jnp.int32, (NBLK_PAD, N_EXPERTS), 0)
    ge = bi >= jnp.broadcast_to(pstart_blk.astype(jnp.int32),
                                (NBLK_PAD, N_EXPERTS))
    bexp_ref[...] = jnp.sum(ge.astype(jnp.int32), axis=1, keepdims=True) - 1
    nblk_ref[...] = jnp.sum(pblk, axis=1, keepdims=True).astype(jnp.int32)


def _router(x, Wr, br):
    return pl.pallas_call(
        _router_body,
        out_shape=[
            jax.ShapeDtypeStruct((TOKENS, TOP_K), jnp.float32),
            jax.ShapeDtypeStruct((TOKENS, TOP_K), jnp.int32),
            jax.ShapeDtypeStruct((NBLK_PAD, 1), jnp.int32),
            jax.ShapeDtypeStruct((1, 1), jnp.int32),
        ],
    )(x, Wr, br.reshape(1, N_EXPERTS))


def _sc_scatter(x, pos_flat):
    mesh = plsc.VectorSubcoreMesh(core_axis_name="core",
                                  subcore_axis_name="subcore")

    @functools.partial(
        pl.kernel,
        out_type=jax.ShapeDtypeStruct((NROWS, N_EMBD), jnp.float32),
        mesh=mesh)
    def run(x_hbm, i_hbm, o_hbm):
        def body(idx, x_vmem, i_vmem):
            j = idx[1]
            pltpu.sync_copy(
                x_vmem,
                o_hbm.at[i_vmem.at[0], pl.ds(j * SC_CHUNK, SC_CHUNK)])

        pltpu.emit_pipeline(
            body,
            grid=(NASSIGN // SC_W, N_EMBD // SC_CHUNK),
            in_specs=[
                pl.BlockSpec((SC_W, SC_CHUNK),
                             index_map=lambda i, j: (i % (TOKENS // SC_W), j)),
                pl.BlockSpec((1, SC_W), index_map=lambda i, j: (0, i)),
            ],
            out_specs=[],
            core_axis_name=("core", "subcore"),
            dimension_semantics=(pltpu.PARALLEL, pltpu.PARALLEL),
            _explicit_indices=True,
        )(x_hbm, i_hbm)

    return run(x, pos_flat)


def _sc_gather(y, pos_flat):
    mesh = plsc.VectorSubcoreMesh(core_axis_name="core",
                                  subcore_axis_name="subcore")

    @functools.partial(
        pl.kernel,
        out_type=jax.ShapeDtypeStruct((NASSIGN, N_EMBD), jnp.float32),
        mesh=mesh)
    def run(y_hbm, i_hbm, o_hbm):
        def body(idx, i_vmem, o_vmem):
            j = idx[1]
            pltpu.sync_copy(
                y_hbm.at[i_vmem.at[0], pl.ds(j * SC_CHUNK, SC_CHUNK)],
                o_vmem)

        pltpu.emit_pipeline(
            body,
            grid=(NASSIGN // SC_W, N_EMBD // SC_CHUNK),
            in_specs=[pl.BlockSpec((1, SC_W), index_map=lambda i, j: (0, i))],
            out_specs=[pl.BlockSpec((SC_W, SC_CHUNK),
                                    index_map=lambda i, j: (i, j))],
            core_axis_name=("core", "subcore"),
            dimension_semantics=(pltpu.PARALLEL, pltpu.PARALLEL),
            _explicit_indices=True,
        )(i_hbm, o_hbm)

    return run(y, pos_flat)


def _ffn_body(bexp_ref, nblk_ref, buf_ref, w1_ref, b1_ref, w2_ref, b2_ref,
              y_ref):
    b = pl.program_id(0)

    @pl.when(b < nblk_ref[0])
    def _():
        h = jnp.dot(buf_ref[...].astype(jnp.bfloat16), w1_ref[0],
                    preferred_element_type=jnp.float32) + b1_ref[0]
        h = 0.5 * h * (1.0 + jax.lax.erf(h * 0.7071067811865476))
        y_ref[...] = jnp.dot(h.astype(jnp.bfloat16), w2_ref[0],
                             preferred_element_type=jnp.float32) + b2_ref[0]


def _grouped_ffn(buf, bexp, nblk, W1b, b1r, W2b, b2r):
    grid_spec = pltpu.PrefetchScalarGridSpec(
        num_scalar_prefetch=2,
        grid=(NBLK,),
        in_specs=[
            pl.BlockSpec((RB, N_EMBD), lambda b, se, nb: (b, 0)),
            pl.BlockSpec((1, N_EMBD, HIDDEN), lambda b, se, nb: (se[b], 0, 0)),
            pl.BlockSpec((1, 1, HIDDEN), lambda b, se, nb: (se[b], 0, 0)),
            pl.BlockSpec((1, HIDDEN, N_EMBD), lambda b, se, nb: (se[b], 0, 0)),
            pl.BlockSpec((1, 1, N_EMBD), lambda b, se, nb: (se[b], 0, 0)),
        ],
        out_specs=pl.BlockSpec((RB, N_EMBD), lambda b, se, nb: (b, 0)),
    )
    return pl.pallas_call(
        _ffn_body,
        grid_spec=grid_spec,
        out_shape=jax.ShapeDtypeStruct((NROWS, N_EMBD), jnp.float32),
    )(bexp, nblk, buf, W1b, b1r, W2b, b2r)


def _combine_body(g0_ref, g1_ref, w01_ref, out_ref):
    out_ref[...] = (w01_ref[:, 0:1] * g0_ref[...]
                    + w01_ref[:, 1:2] * g1_ref[...])


def _combine(g, w01):
    tb = 256
    return pl.pallas_call(
        _combine_body,
        grid=(TOKENS // tb,),
        in_specs=[
            pl.BlockSpec((tb, N_EMBD), lambda t: (t, 0)),
            pl.BlockSpec((tb, N_EMBD), lambda t: (t + TOKENS // tb, 0)),
            pl.BlockSpec((tb, TOP_K), lambda t: (t, 0)),
        ],
        out_specs=pl.BlockSpec((tb, N_EMBD), lambda t: (t, 0)),
        out_shape=jax.ShapeDtypeStruct((TOKENS, N_EMBD), jnp.float32),
    )(g, g, w01)


def kernel(x, Wr, br, W1, b1, W2, b2):
    w01, posk, bexp, nblk = _router(x, Wr, br)
    pos_flat = posk.T.reshape(1, NASSIGN)
    buf = _sc_scatter(x, pos_flat)
    y = _grouped_ffn(buf, bexp.reshape(NBLK_PAD), nblk.reshape(1),
                     W1.astype(jnp.bfloat16),
                     b1.reshape(N_EXPERTS, 1, HIDDEN),
                     W2.astype(jnp.bfloat16),
                     b2.reshape(N_EXPERTS, 1, N_EMBD))
    g = _sc_gather(y, pos_flat)
    return _combine(g, w01)
